# baseline (speedup 1.0000x reference)
import functools

import jax
import jax.numpy as jnp
from jax import lax
from jax.experimental import pallas as pl
from jax.experimental.pallas import tpu as pltpu
from jax.experimental.pallas import tpu_sc as plsc

LANES = 16
NUM_CORES = 2
NUM_SUBCORES = 16
NUM_WORKERS = NUM_CORES * NUM_SUBCORES
CPAD = 17
ISLICE = 128
EPR = ISLICE // 2


def _rsqrt(x):
    i = plsc.bitcast(x, jnp.int32)
    i = 0x5F3759DF - (i >> 1)
    y = plsc.bitcast(i, jnp.float32)
    for _ in range(3):
        y = y * (1.5 - 0.5 * x * y * y)
    return y


def _div9(x):
    return (x.astype(jnp.float32) * jnp.float32(1.0 / 9.0)).astype(jnp.int32)


def _make_sc_kernel(batch, num_cameras, dim, bpw):
    nchunks = bpw // LANES
    nslices = bpw // EPR
    imax = (16 * num_cameras - dim) // dim
    mesh = plsc.VectorSubcoreMesh(core_axis_name="c", subcore_axis_name="s")

    @functools.partial(
        pl.kernel,
        out_type=jax.ShapeDtypeStruct((batch, 16), jnp.float32),
        mesh=mesh,
        scratch_types=[
            pltpu.VMEM((bpw // ISLICE, ISLICE), jnp.int32),
            pltpu.VMEM((nslices, ISLICE), jnp.int32),
            *([pltpu.VMEM((ISLICE, dim), jnp.float32)] * 8),
            pltpu.VMEM((bpw, CPAD), jnp.float32),
            pltpu.VMEM((bpw, CPAD), jnp.float32),
            pltpu.SemaphoreType.DMA,
            pltpu.SemaphoreType.DMA,
        ],
        compiler_params=pltpu.CompilerParams(
            needs_layout_passes=False, use_tc_tiling_on_sc=False),
    )
    def sc_kernel(c2w_hbm, ids_hbm, tab_hbm, out_hbm,
                  idx_v, gidx_v, d0, d1, d2, d3, d4, d5, d6, d7,
                  c2w_v, out_v, sem_g, sem_c):
        dslices = [d0, d1, d2, d3, d4, d5, d6, d7]
        wid = lax.axis_index("s") * NUM_CORES + lax.axis_index("c")
        base = wid * bpw
        nidrows = bpw // ISLICE
        lane = lax.iota(jnp.int32, LANES)

        pltpu.sync_copy(ids_hbm.at[pl.ds(wid * nidrows, nidrows)], idx_v)
        load = pltpu.async_copy(
            c2w_hbm.at[pl.ds(base, bpw)], c2w_v.at[:, pl.ds(0, 16)], sem_c)

        def build(c, carry):
            e = c * LANES + lane
            cam = plsc.load_gather(idx_v, [e >> 7, e & 127])
            i0 = _div9(cam << 4)
            i1 = jnp.minimum(i0 + 1, imax)
            col = (e & (EPR - 1)) << 1
            row = e >> 6
            plsc.store_scatter(gidx_v, [row, col], i0)
            plsc.store_scatter(gidx_v, [row, col + 1], i1)
            return carry

        lax.fori_loop(0, nchunks, build, 0)

        gathers = [
            pltpu.async_copy(tab_hbm.at[gidx_v.at[j]], dslices[j], sem_g)
            for j in range(nslices)
        ]
        for g in gathers:
            g.wait()
        load.wait()

        def body(j):
            dv = dslices[j]

            def chunk(c, carry):
                e = (j * (EPR // LANES) + c) * LANES + lane
                cam = plsc.load_gather(idx_v, [e >> 7, e & 127])
                off = (cam << 4) - ((_div9(cam << 4) << 3) + _div9(cam << 4))
                wbase = ((e & (EPR - 1)) << 4) + ((e & (EPR - 1)) << 1) + off

                d = []
                for k in range(9):
                    w = wbase + k
                    d.append(plsc.load_gather(dv, [w >> 4, w & 15]))
                cw = [
                    plsc.load_gather(
                        c2w_v, [e, jnp.full((LANES,), k, jnp.int32)])
                    for k in range(16)
                ]

                a10, a11, a12 = d[3] + 1.0, d[4], d[5]
                a20, a21, a22 = d[6], d[7] + 1.0, d[8]
                n1 = a10 * a10 + a11 * a11 + a12 * a12
                inv1 = _rsqrt(jnp.maximum(n1, 1e-24))
                b10, b11, b12 = a10 * inv1, a11 * inv1, a12 * inv1
                proj = b10 * a20 + b11 * a21 + b12 * a22
                u0 = a20 - proj * b10
                u1 = a21 - proj * b11
                u2 = a22 - proj * b12
                n2 = u0 * u0 + u1 * u1 + u2 * u2
                inv2 = _rsqrt(jnp.maximum(n2, 1e-24))
                b20, b21, b22 = u0 * inv2, u1 * inv2, u2 * inv2
                b30 = b11 * b22 - b12 * b21
                b31 = b12 * b20 - b10 * b22
                b32 = b10 * b21 - b11 * b20

                t = [[b10, b11, b12, d[0]],
                     [b20, b21, b22, d[1]],
                     [b30, b31, b32, d[2]]]

                for i4 in range(4):
                    c0, c1, c2 = t[0], t[1], t[2]
                    r0, r1, r2 = cw[4 * i4], cw[4 * i4 + 1], cw[4 * i4 + 2]
                    r3 = cw[4 * i4 + 3]
                    for jj in range(4):
                        v = r0 * c0[jj] + r1 * c1[jj] + r2 * c2[jj]
                        if jj == 3:
                            v = v + r3
                        plsc.store_scatter(
                            out_v,
                            [e, jnp.full((LANES,), 4 * i4 + jj, jnp.int32)],
                            v)
                return carry

            lax.fori_loop(0, EPR // LANES, chunk, 0)

        for j in range(nslices):
            body(j)

        pltpu.sync_copy(out_v.at[:, pl.ds(0, 16)],
                        out_hbm.at[pl.ds(base, bpw)])

    return sc_kernel


def kernel(camtoworlds, camera_ids, embeds_weight):
    batch = camtoworlds.shape[0]
    bpw = batch // NUM_WORKERS
    num_cameras, dim = embeds_weight.shape
    c2w = camtoworlds.reshape(batch, 16)
    ids2 = camera_ids.reshape(batch // ISLICE, ISLICE)
    sc = _make_sc_kernel(batch, num_cameras, dim, bpw)
    out = sc(c2w, ids2, embeds_weight)
    return out.reshape(batch, 4, 4)
